# Initial kernel scaffold; baseline (speedup 1.0000x reference)
import jax
import jax.numpy as jnp
from jax.experimental import pallas as pl


def kernel(adj_u1, adj_u2, adj_i1, adj_i2, adj_cat, user_emb, item_emb):
    raise NotImplementedError("write your pallas kernel here")


# speedup vs baseline: 1.0894x; 1.0894x over previous
import jax
import jax.numpy as jnp
from jax.experimental import pallas as pl


def _pair_mm_body(a_u_ref, a_i_ref, x_u_ref, x_i_ref, o_u_ref, o_i_ref):
    o_u_ref[...] = jnp.dot(a_u_ref[...], x_u_ref[...],
                           preferred_element_type=jnp.float32)
    o_i_ref[...] = jnp.dot(a_i_ref[...], x_i_ref[...],
                           preferred_element_type=jnp.float32)


def _pair_mm(a_u, a_i, x_u, x_i, m_blk):
    m, k = a_u.shape
    n = x_u.shape[1]
    return pl.pallas_call(
        _pair_mm_body,
        grid=(m // m_blk,),
        in_specs=[
            pl.BlockSpec((m_blk, k), lambda i: (i, 0)),
            pl.BlockSpec((m_blk, k), lambda i: (i, 0)),
            pl.BlockSpec((k, n), lambda i: (0, 0)),
            pl.BlockSpec((k, n), lambda i: (0, 0)),
        ],
        out_specs=[
            pl.BlockSpec((m_blk, n), lambda i: (i, 0)),
            pl.BlockSpec((m_blk, n), lambda i: (i, 0)),
        ],
        out_shape=[
            jax.ShapeDtypeStruct((m, n), jnp.float32),
            jax.ShapeDtypeStruct((m, n), jnp.float32),
        ],
    )(a_u, a_i, x_u, x_i)


def _cat_mm_body(c_ref, t_u_ref, t_i_ref, o_u_ref, o_i_ref):
    c = c_ref[...]
    o_u_ref[...] = jnp.dot(c, t_u_ref[...], preferred_element_type=jnp.float32)
    o_i_ref[...] = jnp.dot(c, t_i_ref[...], preferred_element_type=jnp.float32)


def _cat_mm(adj_cat, t_u, t_i, m_blk):
    m, k = adj_cat.shape
    n = t_u.shape[1]
    return pl.pallas_call(
        _cat_mm_body,
        grid=(m // m_blk,),
        in_specs=[
            pl.BlockSpec((m_blk, k), lambda i: (i, 0)),
            pl.BlockSpec((k, n), lambda i: (0, 0)),
            pl.BlockSpec((k, n), lambda i: (0, 0)),
        ],
        out_specs=[
            pl.BlockSpec((m_blk, n), lambda i: (i, 0)),
            pl.BlockSpec((m_blk, n), lambda i: (i, 0)),
        ],
        out_shape=[
            jax.ShapeDtypeStruct((m, n), jnp.float32),
            jax.ShapeDtypeStruct((m, n), jnp.float32),
        ],
    )(adj_cat, t_u, t_i)


def _cat_mean_body(c_ref, t_u_ref, t_i_ref, eu0_ref, eu1_ref, ei0_ref,
                   ei1_ref, o_u_ref, o_i_ref):
    c = c_ref[...]
    inv = jnp.float32(1.0 / 3.0)
    o_u_ref[...] = (eu0_ref[...] + eu1_ref[...] +
                    jnp.dot(c, t_u_ref[...],
                            preferred_element_type=jnp.float32)) * inv
    o_i_ref[...] = (ei0_ref[...] + ei1_ref[...] +
                    jnp.dot(c, t_i_ref[...],
                            preferred_element_type=jnp.float32)) * inv


def _cat_mm_mean(adj_cat, t_u, t_i, e_u0, e_u1, e_i0, e_i1, m_blk):
    m, k = adj_cat.shape
    n = t_u.shape[1]
    row_spec = pl.BlockSpec((m_blk, n), lambda i: (i, 0))
    return pl.pallas_call(
        _cat_mean_body,
        grid=(m // m_blk,),
        in_specs=[
            pl.BlockSpec((m_blk, k), lambda i: (i, 0)),
            pl.BlockSpec((k, n), lambda i: (0, 0)),
            pl.BlockSpec((k, n), lambda i: (0, 0)),
            row_spec, row_spec, row_spec, row_spec,
        ],
        out_specs=[row_spec, row_spec],
        out_shape=[
            jax.ShapeDtypeStruct((m, n), jnp.float32),
            jax.ShapeDtypeStruct((m, n), jnp.float32),
        ],
    )(adj_cat, t_u, t_i, e_u0, e_u1, e_i0, e_i1)


_M_BLK_HYPER_DOWN = 128
_M_BLK_HYPER_UP = 400
_M_BLK_CAT = 400


def kernel(adj_u1, adj_u2, adj_i1, adj_i2, adj_cat, user_emb, item_emb):
    e_u0, e_i0 = user_emb, item_emb

    s_u, s_i = _pair_mm(adj_u2, adj_i2, e_u0, e_i0, _M_BLK_HYPER_DOWN)
    t_u, t_i = _pair_mm(adj_u1, adj_i1, s_u, s_i, _M_BLK_HYPER_UP)
    e_u1, e_i1 = _cat_mm(adj_cat, t_u, t_i, _M_BLK_CAT)

    s_u, s_i = _pair_mm(adj_u2, adj_i2, e_u1, e_i1, _M_BLK_HYPER_DOWN)
    t_u, t_i = _pair_mm(adj_u1, adj_i1, s_u, s_i, _M_BLK_HYPER_UP)
    u_emb, i_emb = _cat_mm_mean(adj_cat, t_u, t_i, e_u0, e_u1, e_i0, e_i1,
                                _M_BLK_CAT)
    return (u_emb, i_emb)


# speedup vs baseline: 1.1157x; 1.0241x over previous
import jax
import jax.numpy as jnp
from jax.experimental import pallas as pl


def _pair_mm_body(a_u_ref, a_i_ref, x_u_ref, x_i_ref, o_u_ref, o_i_ref):
    o_u_ref[...] = jnp.dot(a_u_ref[...], x_u_ref[...],
                           preferred_element_type=jnp.float32)
    o_i_ref[...] = jnp.dot(a_i_ref[...], x_i_ref[...],
                           preferred_element_type=jnp.float32)


def _pair_mm(a_u, a_i, x_u, x_i, m_blk):
    m, k = a_u.shape
    n = x_u.shape[1]
    return pl.pallas_call(
        _pair_mm_body,
        grid=(m // m_blk,),
        in_specs=[
            pl.BlockSpec((m_blk, k), lambda i: (i, 0)),
            pl.BlockSpec((m_blk, k), lambda i: (i, 0)),
            pl.BlockSpec((k, n), lambda i: (0, 0)),
            pl.BlockSpec((k, n), lambda i: (0, 0)),
        ],
        out_specs=[
            pl.BlockSpec((m_blk, n), lambda i: (i, 0)),
            pl.BlockSpec((m_blk, n), lambda i: (i, 0)),
        ],
        out_shape=[
            jax.ShapeDtypeStruct((m, n), jnp.float32),
            jax.ShapeDtypeStruct((m, n), jnp.float32),
        ],
    )(a_u, a_i, x_u, x_i)


def _pair_mm_cat_body(a_u_ref, a_i_ref, x_u_ref, x_i_ref, o_ref):
    t_u = jnp.dot(a_u_ref[...], x_u_ref[...],
                  preferred_element_type=jnp.float32)
    t_i = jnp.dot(a_i_ref[...], x_i_ref[...],
                  preferred_element_type=jnp.float32)
    o_ref[...] = jnp.concatenate([t_u, t_i], axis=-1)


def _pair_mm_cat(a_u, a_i, x_u, x_i, m_blk):
    m, k = a_u.shape
    n = x_u.shape[1]
    return pl.pallas_call(
        _pair_mm_cat_body,
        grid=(m // m_blk,),
        in_specs=[
            pl.BlockSpec((m_blk, k), lambda i: (i, 0)),
            pl.BlockSpec((m_blk, k), lambda i: (i, 0)),
            pl.BlockSpec((k, n), lambda i: (0, 0)),
            pl.BlockSpec((k, n), lambda i: (0, 0)),
        ],
        out_specs=pl.BlockSpec((m_blk, 2 * n), lambda i: (i, 0)),
        out_shape=jax.ShapeDtypeStruct((m, 2 * n), jnp.float32),
    )(a_u, a_i, x_u, x_i)


def _cat_mm_body(c_ref, t_ref, o_u_ref, o_i_ref):
    n = o_u_ref.shape[1]
    r = jnp.dot(c_ref[...], t_ref[...], preferred_element_type=jnp.float32)
    o_u_ref[...] = r[:, :n]
    o_i_ref[...] = r[:, n:]


def _cat_mm(adj_cat, t_cat, m_blk):
    m, k = adj_cat.shape
    n = t_cat.shape[1] // 2
    return pl.pallas_call(
        _cat_mm_body,
        grid=(m // m_blk,),
        in_specs=[
            pl.BlockSpec((m_blk, k), lambda i: (i, 0)),
            pl.BlockSpec((k, 2 * n), lambda i: (0, 0)),
        ],
        out_specs=[
            pl.BlockSpec((m_blk, n), lambda i: (i, 0)),
            pl.BlockSpec((m_blk, n), lambda i: (i, 0)),
        ],
        out_shape=[
            jax.ShapeDtypeStruct((m, n), jnp.float32),
            jax.ShapeDtypeStruct((m, n), jnp.float32),
        ],
    )(adj_cat, t_cat)


def _cat_mean_body(c_ref, t_ref, eu0_ref, eu1_ref, ei0_ref, ei1_ref,
                   o_u_ref, o_i_ref):
    n = o_u_ref.shape[1]
    r = jnp.dot(c_ref[...], t_ref[...], preferred_element_type=jnp.float32)
    inv = jnp.float32(1.0 / 3.0)
    o_u_ref[...] = (eu0_ref[...] + eu1_ref[...] + r[:, :n]) * inv
    o_i_ref[...] = (ei0_ref[...] + ei1_ref[...] + r[:, n:]) * inv


def _cat_mm_mean(adj_cat, t_cat, e_u0, e_u1, e_i0, e_i1, m_blk):
    m, k = adj_cat.shape
    n = t_cat.shape[1] // 2
    row_spec = pl.BlockSpec((m_blk, n), lambda i: (i, 0))
    return pl.pallas_call(
        _cat_mean_body,
        grid=(m // m_blk,),
        in_specs=[
            pl.BlockSpec((m_blk, k), lambda i: (i, 0)),
            pl.BlockSpec((k, 2 * n), lambda i: (0, 0)),
            row_spec, row_spec, row_spec, row_spec,
        ],
        out_specs=[row_spec, row_spec],
        out_shape=[
            jax.ShapeDtypeStruct((m, n), jnp.float32),
            jax.ShapeDtypeStruct((m, n), jnp.float32),
        ],
    )(adj_cat, t_cat, e_u0, e_u1, e_i0, e_i1)


_M_BLK_HYPER_DOWN = 128
_M_BLK_HYPER_UP = 400
_M_BLK_CAT = 400


def kernel(adj_u1, adj_u2, adj_i1, adj_i2, adj_cat, user_emb, item_emb):
    e_u0, e_i0 = user_emb, item_emb

    s_u, s_i = _pair_mm(adj_u2, adj_i2, e_u0, e_i0, _M_BLK_HYPER_DOWN)
    t_cat = _pair_mm_cat(adj_u1, adj_i1, s_u, s_i, _M_BLK_HYPER_UP)
    e_u1, e_i1 = _cat_mm(adj_cat, t_cat, _M_BLK_CAT)

    s_u, s_i = _pair_mm(adj_u2, adj_i2, e_u1, e_i1, _M_BLK_HYPER_DOWN)
    t_cat = _pair_mm_cat(adj_u1, adj_i1, s_u, s_i, _M_BLK_HYPER_UP)
    u_emb, i_emb = _cat_mm_mean(adj_cat, t_cat, e_u0, e_u1, e_i0, e_i1,
                                _M_BLK_CAT)
    return (u_emb, i_emb)


# speedup vs baseline: 1.4413x; 1.2918x over previous
import jax
import jax.numpy as jnp
from jax.experimental import pallas as pl

_CONTRACT_ROWS = (((0,), (0,)), ((), ()))


def _pair_tmm_body(a_u_ref, a_i_ref, x_u_ref, x_i_ref, o_u_ref, o_i_ref):
    @pl.when(pl.program_id(0) == 0)
    def _init():
        o_u_ref[...] = jnp.zeros_like(o_u_ref)
        o_i_ref[...] = jnp.zeros_like(o_i_ref)

    o_u_ref[...] += jax.lax.dot_general(a_u_ref[...], x_u_ref[...],
                                        _CONTRACT_ROWS,
                                        preferred_element_type=jnp.float32)
    o_i_ref[...] += jax.lax.dot_general(a_i_ref[...], x_i_ref[...],
                                        _CONTRACT_ROWS,
                                        preferred_element_type=jnp.float32)


def _pair_tmm(a_u, a_i, x_u, x_i, r_blk):
    k, m = a_u.shape
    n = x_u.shape[1]
    return pl.pallas_call(
        _pair_tmm_body,
        grid=(k // r_blk,),
        in_specs=[
            pl.BlockSpec((r_blk, m), lambda i: (i, 0)),
            pl.BlockSpec((r_blk, m), lambda i: (i, 0)),
            pl.BlockSpec((r_blk, n), lambda i: (i, 0)),
            pl.BlockSpec((r_blk, n), lambda i: (i, 0)),
        ],
        out_specs=[
            pl.BlockSpec((m, n), lambda i: (0, 0)),
            pl.BlockSpec((m, n), lambda i: (0, 0)),
        ],
        out_shape=[
            jax.ShapeDtypeStruct((m, n), jnp.float32),
            jax.ShapeDtypeStruct((m, n), jnp.float32),
        ],
    )(a_u, a_i, x_u, x_i)


def _pair_mm_cat_body(a_u_ref, a_i_ref, x_u_ref, x_i_ref, o_ref):
    t_u = jnp.dot(a_u_ref[...], x_u_ref[...],
                  preferred_element_type=jnp.float32)
    t_i = jnp.dot(a_i_ref[...], x_i_ref[...],
                  preferred_element_type=jnp.float32)
    o_ref[...] = jnp.concatenate([t_u, t_i], axis=-1)


def _pair_mm_cat(a_u, a_i, x_u, x_i, m_blk):
    m, k = a_u.shape
    n = x_u.shape[1]
    return pl.pallas_call(
        _pair_mm_cat_body,
        grid=(m // m_blk,),
        in_specs=[
            pl.BlockSpec((m_blk, k), lambda i: (i, 0)),
            pl.BlockSpec((m_blk, k), lambda i: (i, 0)),
            pl.BlockSpec((k, n), lambda i: (0, 0)),
            pl.BlockSpec((k, n), lambda i: (0, 0)),
        ],
        out_specs=pl.BlockSpec((m_blk, 2 * n), lambda i: (i, 0)),
        out_shape=jax.ShapeDtypeStruct((m, 2 * n), jnp.float32),
    )(a_u, a_i, x_u, x_i)


def _cat_mm_body(c_ref, t_ref, o_u_ref, o_i_ref):
    n = o_u_ref.shape[1]
    r = jnp.dot(c_ref[...], t_ref[...], preferred_element_type=jnp.float32)
    o_u_ref[...] = r[:, :n]
    o_i_ref[...] = r[:, n:]


def _cat_mm(adj_cat, t_cat, m_blk):
    m, k = adj_cat.shape
    n = t_cat.shape[1] // 2
    return pl.pallas_call(
        _cat_mm_body,
        grid=(m // m_blk,),
        in_specs=[
            pl.BlockSpec((m_blk, k), lambda i: (i, 0)),
            pl.BlockSpec((k, 2 * n), lambda i: (0, 0)),
        ],
        out_specs=[
            pl.BlockSpec((m_blk, n), lambda i: (i, 0)),
            pl.BlockSpec((m_blk, n), lambda i: (i, 0)),
        ],
        out_shape=[
            jax.ShapeDtypeStruct((m, n), jnp.float32),
            jax.ShapeDtypeStruct((m, n), jnp.float32),
        ],
    )(adj_cat, t_cat)


def _cat_mean_body(c_ref, t_ref, eu0_ref, eu1_ref, ei0_ref, ei1_ref,
                   o_u_ref, o_i_ref):
    n = o_u_ref.shape[1]
    r = jnp.dot(c_ref[...], t_ref[...], preferred_element_type=jnp.float32)
    inv = jnp.float32(1.0 / 3.0)
    o_u_ref[...] = (eu0_ref[...] + eu1_ref[...] + r[:, :n]) * inv
    o_i_ref[...] = (ei0_ref[...] + ei1_ref[...] + r[:, n:]) * inv


def _cat_mm_mean(adj_cat, t_cat, e_u0, e_u1, e_i0, e_i1, m_blk):
    m, k = adj_cat.shape
    n = t_cat.shape[1] // 2
    row_spec = pl.BlockSpec((m_blk, n), lambda i: (i, 0))
    return pl.pallas_call(
        _cat_mean_body,
        grid=(m // m_blk,),
        in_specs=[
            pl.BlockSpec((m_blk, k), lambda i: (i, 0)),
            pl.BlockSpec((k, 2 * n), lambda i: (0, 0)),
            row_spec, row_spec, row_spec, row_spec,
        ],
        out_specs=[row_spec, row_spec],
        out_shape=[
            jax.ShapeDtypeStruct((m, n), jnp.float32),
            jax.ShapeDtypeStruct((m, n), jnp.float32),
        ],
    )(adj_cat, t_cat, e_u0, e_u1, e_i0, e_i1)


_R_BLK_HYPER_DOWN = 1000
_M_BLK_HYPER_UP = 1000
_M_BLK_CAT = 400


def kernel(adj_u1, adj_u2, adj_i1, adj_i2, adj_cat, user_emb, item_emb):
    e_u0, e_i0 = user_emb, item_emb
    a2t_u, a2t_i = adj_u2.T, adj_i2.T

    s_u, s_i = _pair_tmm(a2t_u, a2t_i, e_u0, e_i0, _R_BLK_HYPER_DOWN)
    t_cat = _pair_mm_cat(adj_u1, adj_i1, s_u, s_i, _M_BLK_HYPER_UP)
    e_u1, e_i1 = _cat_mm(adj_cat, t_cat, _M_BLK_CAT)

    s_u, s_i = _pair_tmm(a2t_u, a2t_i, e_u1, e_i1, _R_BLK_HYPER_DOWN)
    t_cat = _pair_mm_cat(adj_u1, adj_i1, s_u, s_i, _M_BLK_HYPER_UP)
    u_emb, i_emb = _cat_mm_mean(adj_cat, t_cat, e_u0, e_u1, e_i0, e_i1,
                                _M_BLK_CAT)
    return (u_emb, i_emb)


# speedup vs baseline: 1.4472x; 1.0041x over previous
import jax
import jax.numpy as jnp
from jax.experimental import pallas as pl

_CONTRACT_ROWS = (((0,), (0,)), ((), ()))


def _pair_tmm_body(a_u_ref, a_i_ref, x_u_ref, x_i_ref, o_u_ref, o_i_ref):
    @pl.when(pl.program_id(0) == 0)
    def _init():
        o_u_ref[...] = jnp.zeros_like(o_u_ref)
        o_i_ref[...] = jnp.zeros_like(o_i_ref)

    o_u_ref[...] += jax.lax.dot_general(a_u_ref[...], x_u_ref[...],
                                        _CONTRACT_ROWS,
                                        preferred_element_type=jnp.float32)
    o_i_ref[...] += jax.lax.dot_general(a_i_ref[...], x_i_ref[...],
                                        _CONTRACT_ROWS,
                                        preferred_element_type=jnp.float32)


def _pair_tmm(a_u, a_i, x_u, x_i, r_blk):
    k, m = a_u.shape
    n = x_u.shape[1]
    return pl.pallas_call(
        _pair_tmm_body,
        grid=(k // r_blk,),
        in_specs=[
            pl.BlockSpec((r_blk, m), lambda i: (i, 0)),
            pl.BlockSpec((r_blk, m), lambda i: (i, 0)),
            pl.BlockSpec((r_blk, n), lambda i: (i, 0)),
            pl.BlockSpec((r_blk, n), lambda i: (i, 0)),
        ],
        out_specs=[
            pl.BlockSpec((m, n), lambda i: (0, 0)),
            pl.BlockSpec((m, n), lambda i: (0, 0)),
        ],
        out_shape=[
            jax.ShapeDtypeStruct((m, n), jnp.float32),
            jax.ShapeDtypeStruct((m, n), jnp.float32),
        ],
    )(a_u, a_i, x_u, x_i)


def _pair_mm_cat_body(a_u_ref, a_i_ref, x_u_ref, x_i_ref, o_ref):
    t_u = jnp.dot(a_u_ref[...], x_u_ref[...],
                  preferred_element_type=jnp.float32)
    t_i = jnp.dot(a_i_ref[...], x_i_ref[...],
                  preferred_element_type=jnp.float32)
    o_ref[...] = jnp.concatenate([t_u, t_i], axis=-1)


def _pair_mm_cat(a_u, a_i, x_u, x_i, m_blk):
    m, k = a_u.shape
    n = x_u.shape[1]
    return pl.pallas_call(
        _pair_mm_cat_body,
        grid=(m // m_blk,),
        in_specs=[
            pl.BlockSpec((m_blk, k), lambda i: (i, 0)),
            pl.BlockSpec((m_blk, k), lambda i: (i, 0)),
            pl.BlockSpec((k, n), lambda i: (0, 0)),
            pl.BlockSpec((k, n), lambda i: (0, 0)),
        ],
        out_specs=pl.BlockSpec((m_blk, 2 * n), lambda i: (i, 0)),
        out_shape=jax.ShapeDtypeStruct((m, 2 * n), jnp.float32),
    )(a_u, a_i, x_u, x_i)


def _cat_mm_body(c_ref, t_ref, o_u_ref, o_i_ref):
    n = o_u_ref.shape[1]
    r = jnp.dot(c_ref[...], t_ref[...], preferred_element_type=jnp.float32)
    o_u_ref[...] = r[:, :n]
    o_i_ref[...] = r[:, n:]


def _cat_mm(adj_cat, t_cat, m_blk):
    m, k = adj_cat.shape
    n = t_cat.shape[1] // 2
    return pl.pallas_call(
        _cat_mm_body,
        grid=(m // m_blk,),
        in_specs=[
            pl.BlockSpec((m_blk, k), lambda i: (i, 0)),
            pl.BlockSpec((k, 2 * n), lambda i: (0, 0)),
        ],
        out_specs=[
            pl.BlockSpec((m_blk, n), lambda i: (i, 0)),
            pl.BlockSpec((m_blk, n), lambda i: (i, 0)),
        ],
        out_shape=[
            jax.ShapeDtypeStruct((m, n), jnp.float32),
            jax.ShapeDtypeStruct((m, n), jnp.float32),
        ],
    )(adj_cat, t_cat)


def _cat_mean_body(c_ref, t_ref, eu0_ref, eu1_ref, ei0_ref, ei1_ref,
                   o_u_ref, o_i_ref):
    n = o_u_ref.shape[1]
    r = jnp.dot(c_ref[...], t_ref[...], preferred_element_type=jnp.float32)
    inv = jnp.float32(1.0 / 3.0)
    o_u_ref[...] = (eu0_ref[...] + eu1_ref[...] + r[:, :n]) * inv
    o_i_ref[...] = (ei0_ref[...] + ei1_ref[...] + r[:, n:]) * inv


def _cat_mm_mean(adj_cat, t_cat, e_u0, e_u1, e_i0, e_i1, m_blk):
    m, k = adj_cat.shape
    n = t_cat.shape[1] // 2
    row_spec = pl.BlockSpec((m_blk, n), lambda i: (i, 0))
    return pl.pallas_call(
        _cat_mean_body,
        grid=(m // m_blk,),
        in_specs=[
            pl.BlockSpec((m_blk, k), lambda i: (i, 0)),
            pl.BlockSpec((k, 2 * n), lambda i: (0, 0)),
            row_spec, row_spec, row_spec, row_spec,
        ],
        out_specs=[row_spec, row_spec],
        out_shape=[
            jax.ShapeDtypeStruct((m, n), jnp.float32),
            jax.ShapeDtypeStruct((m, n), jnp.float32),
        ],
    )(adj_cat, t_cat, e_u0, e_u1, e_i0, e_i1)


_R_BLK_HYPER_DOWN = 1000
_M_BLK_HYPER_UP = 1000
_M_BLK_CAT = 200


def kernel(adj_u1, adj_u2, adj_i1, adj_i2, adj_cat, user_emb, item_emb):
    e_u0, e_i0 = user_emb, item_emb
    a2t_u, a2t_i = adj_u2.T, adj_i2.T

    s_u, s_i = _pair_tmm(a2t_u, a2t_i, e_u0, e_i0, _R_BLK_HYPER_DOWN)
    t_cat = _pair_mm_cat(adj_u1, adj_i1, s_u, s_i, _M_BLK_HYPER_UP)
    e_u1, e_i1 = _cat_mm(adj_cat, t_cat, _M_BLK_CAT)

    s_u, s_i = _pair_tmm(a2t_u, a2t_i, e_u1, e_i1, _R_BLK_HYPER_DOWN)
    t_cat = _pair_mm_cat(adj_u1, adj_i1, s_u, s_i, _M_BLK_HYPER_UP)
    u_emb, i_emb = _cat_mm_mean(adj_cat, t_cat, e_u0, e_u1, e_i0, e_i1,
                                _M_BLK_CAT)
    return (u_emb, i_emb)
